# baseline (speedup 1.0000x reference)
import functools

import jax
import jax.numpy as jnp
from jax import lax
from jax.experimental import pallas as pl
from jax.experimental.pallas import tpu as pltpu
from jax.experimental.pallas import tpu_sc as plsc

B = 4096
NUM_TABLES = 26
VOCAB = 100000
DIM = 32
NUM_FEATS = NUM_TABLES + 1
Z_PAD = 384

SC_CORES = 2
SC_SUBCORES = 16
NW = SC_CORES * SC_SUBCORES
SPLIT = 2
T_HALF = NUM_TABLES // SPLIT
N_PAIRS = T_HALF * DIM
PAIRS_PER_W = N_PAIRS // NW


def _sc_gather(tab_T, idx_T):
  mesh = plsc.VectorSubcoreMesh(core_axis_name="c", subcore_axis_name="s")

  @functools.partial(
      pl.kernel,
      mesh=mesh,
      compiler_params=pltpu.CompilerParams(
          use_tc_tiling_on_sc=False, needs_layout_passes=False),
      out_type=jax.ShapeDtypeStruct((N_PAIRS, B), jnp.float32),
      scratch_types=[
          pltpu.VMEM((VOCAB,), jnp.float32),
          pltpu.VMEM((B,), jnp.int32),
          pltpu.VMEM((B,), jnp.float32),
          pltpu.SemaphoreType.DMA,
      ],
  )
  def gather_kernel(tab_hbm, idx_hbm, out_hbm, row_v, idx_v, out_v, sem):
    wid = lax.axis_index("s") * SC_CORES + lax.axis_index("c")
    p0 = wid * PAIRS_PER_W

    def pair_body(j, carry):
      p = p0 + j
      t = p // DIM
      d = p % DIM
      pltpu.sync_copy(idx_hbm.at[t], idx_v)
      pltpu.sync_copy(tab_hbm.at[t, d], row_v)

      def g_body(i, c):
        iv = idx_v[pl.ds(i * 16, 16)]
        out_v[pl.ds(i * 16, 16)] = plsc.load_gather(row_v, [iv])
        return c

      lax.fori_loop(0, B // 16, g_body, 0, unroll=4)
      pltpu.sync_copy(out_v, out_hbm.at[p])
      return carry

    lax.fori_loop(0, PAIRS_PER_W, pair_body, 0)

  return gather_kernel(tab_T, idx_T)


def _tc_body(num_ref, emb_ref, wb0, bb0, wb1, bb1, wb2, bb2,
             wt0, bt0, wt1, bt1, wt2, bt2, wt3, bt3, wt4, bt4,
             out_ref, z_scr):
  mm = lambda w, x: jnp.dot(w, x, preferred_element_type=jnp.float32)
  x = num_ref[:]
  x = jax.nn.relu(mm(wb0[:], x) + bb0[:])
  x = jax.nn.relu(mm(wb1[:], x) + bb1[:])
  x = jax.nn.relu(mm(wb2[:], x) + bb2[:])

  feats = [x] + [emb_ref[pl.ds(DIM * t, DIM), :] for t in range(NUM_TABLES)]

  z_scr[0:DIM, :] = x
  row = DIM
  for i in range(1, NUM_FEATS):
    fi = feats[i]
    for jj in range(i):
      z_scr[pl.ds(row, 1), :] = jnp.sum(fi * feats[jj], axis=0)[None, :]
      row += 1
  z_scr[pl.ds(Z_PAD - 1, 1), :] = jnp.zeros_like(z_scr[pl.ds(Z_PAD - 1, 1), :])

  z = z_scr[:]
  z = jax.nn.relu(mm(wt0[:], z) + bt0[:])
  z = jax.nn.relu(mm(wt1[:], z) + bt1[:])
  z = jax.nn.relu(mm(wt2[:], z) + bt2[:])
  z = jax.nn.relu(mm(wt3[:], z) + bt3[:])
  out_ref[:] = mm(wt4[:], z) + bt4[:]


def _tc_dense(num_T, pairs, Wb0T, bb0, Wb1T, bb1, Wb2T, bb2,
              Wt0T, bt0, Wt1T, bt1, Wt2T, bt2, Wt3T, bt3, Wt4T, bt4, block_b):
  grid = B // block_b
  full2 = lambda w: pl.BlockSpec(w.shape, lambda i: (0, 0))
  in_specs = [
      pl.BlockSpec((num_T.shape[0], block_b), lambda i: (0, i)),
      pl.BlockSpec((NUM_TABLES * DIM, block_b), lambda i: (0, i)),
      full2(Wb0T), full2(bb0), full2(Wb1T), full2(bb1), full2(Wb2T), full2(bb2),
      full2(Wt0T), full2(bt0), full2(Wt1T), full2(bt1), full2(Wt2T), full2(bt2),
      full2(Wt3T), full2(bt3), full2(Wt4T), full2(bt4),
  ]
  return pl.pallas_call(
      _tc_body,
      grid=(grid,),
      in_specs=in_specs,
      out_specs=pl.BlockSpec((1, block_b), lambda i: (0, i)),
      out_shape=jax.ShapeDtypeStruct((1, B), jnp.float32),
      scratch_shapes=[
          pltpu.VMEM((Z_PAD, block_b), jnp.float32),
      ],
  )(num_T, pairs, Wb0T, bb0, Wb1T, bb1, Wb2T, bb2,
    Wt0T, bt0, Wt1T, bt1, Wt2T, bt2, Wt3T, bt3, Wt4T, bt4)


def kernel(numerical_features, categorical_features, embedding_tables,
           Wb0, bb0, Wb1, bb1, Wb2, bb2,
           Wt0, bt0, Wt1, bt1, Wt2, bt2, Wt3, bt3, Wt4, bt4):
  idx_T = categorical_features.T
  halves = [
      _sc_gather(
          embedding_tables[h * T_HALF:(h + 1) * T_HALF].transpose(0, 2, 1),
          idx_T[h * T_HALF:(h + 1) * T_HALF])
      for h in range(SPLIT)
  ]
  pairs = jnp.concatenate(halves, axis=0)

  Wt0p = jnp.concatenate([Wt0, jnp.zeros((1, Wt0.shape[1]), Wt0.dtype)], axis=0)
  col = lambda b: b[:, None]
  out_T = _tc_dense(
      numerical_features.T, pairs,
      Wb0.T, col(bb0), Wb1.T, col(bb1), Wb2.T, col(bb2),
      Wt0p.T, col(bt0), Wt1.T, col(bt1), Wt2.T, col(bt2),
      Wt3.T, col(bt3), Wt4.T, col(bt4),
      block_b=256)
  return out_T.T


# speedup vs baseline: 1.1828x; 1.1828x over previous
import functools

import jax
import jax.numpy as jnp
from jax import lax
from jax.experimental import pallas as pl
from jax.experimental.pallas import tpu as pltpu
from jax.experimental.pallas import tpu_sc as plsc

B = 4096
NUM_TABLES = 26
VOCAB = 100000
DIM = 32
NUM_FEATS = NUM_TABLES + 1
Z_PAD = 384

SC_CORES = 2
SC_SUBCORES = 16
NW = SC_CORES * SC_SUBCORES
N_PAIRS = NUM_TABLES * DIM
PAIRS_PER_W = N_PAIRS // NW


def _sc_gather(tab_T, idx_T):
  mesh = plsc.VectorSubcoreMesh(core_axis_name="c", subcore_axis_name="s")

  @functools.partial(
      pl.kernel,
      mesh=mesh,
      compiler_params=pltpu.CompilerParams(
          use_tc_tiling_on_sc=False, needs_layout_passes=False),
      out_type=jax.ShapeDtypeStruct((N_PAIRS, B), jnp.float32),
      scratch_types=[
          pltpu.VMEM((VOCAB,), jnp.float32),
          pltpu.VMEM((B,), jnp.int32),
          pltpu.VMEM((B,), jnp.float32),
          pltpu.SemaphoreType.DMA,
      ],
  )
  def gather_kernel(tab_hbm, idx_hbm, out_hbm, row_v, idx_v, out_v, sem):
    wid = lax.axis_index("s") * SC_CORES + lax.axis_index("c")
    p0 = wid * PAIRS_PER_W

    def pair_body(j, carry):
      p = p0 + j
      t = p // DIM
      d = p % DIM
      pltpu.sync_copy(idx_hbm.at[t], idx_v)
      pltpu.sync_copy(tab_hbm.at[t, d], row_v)

      def g_body(i, c):
        iv = idx_v[pl.ds(i * 16, 16)]
        out_v[pl.ds(i * 16, 16)] = plsc.load_gather(row_v, [iv])
        return c

      lax.fori_loop(0, B // 16, g_body, 0, unroll=4)
      pltpu.sync_copy(out_v, out_hbm.at[p])
      return carry

    lax.fori_loop(0, PAIRS_PER_W, pair_body, 0)

  return gather_kernel(tab_T, idx_T)


def _tc_body(num_ref, emb_ref, wb0, bb0, wb1, bb1, wb2, bb2,
             wt0, bt0, wt1, bt1, wt2, bt2, wt3, bt3, wt4, bt4,
             out_ref, z_scr):
  mm = lambda w, x: jnp.dot(w, x, preferred_element_type=jnp.float32)
  x = num_ref[:]
  x = jax.nn.relu(mm(wb0[:], x) + bb0[:])
  x = jax.nn.relu(mm(wb1[:], x) + bb1[:])
  x = jax.nn.relu(mm(wb2[:], x) + bb2[:])

  feats = [x] + [emb_ref[pl.ds(DIM * t, DIM), :] for t in range(NUM_TABLES)]

  z_scr[0:DIM, :] = x
  row = DIM
  for i in range(1, NUM_FEATS):
    fi = feats[i]
    for jj in range(i):
      z_scr[pl.ds(row, 1), :] = jnp.sum(fi * feats[jj], axis=0)[None, :]
      row += 1
  z_scr[pl.ds(Z_PAD - 1, 1), :] = jnp.zeros_like(z_scr[pl.ds(Z_PAD - 1, 1), :])

  z = z_scr[:]
  z = jax.nn.relu(mm(wt0[:], z) + bt0[:])
  z = jax.nn.relu(mm(wt1[:], z) + bt1[:])
  z = jax.nn.relu(mm(wt2[:], z) + bt2[:])
  z = jax.nn.relu(mm(wt3[:], z) + bt3[:])
  out_ref[:] = mm(wt4[:], z) + bt4[:]


def _tc_dense(num_T, pairs, Wb0T, bb0, Wb1T, bb1, Wb2T, bb2,
              Wt0T, bt0, Wt1T, bt1, Wt2T, bt2, Wt3T, bt3, Wt4T, bt4, block_b):
  grid = B // block_b
  full2 = lambda w: pl.BlockSpec(w.shape, lambda i: (0, 0))
  in_specs = [
      pl.BlockSpec((num_T.shape[0], block_b), lambda i: (0, i)),
      pl.BlockSpec((NUM_TABLES * DIM, block_b), lambda i: (0, i)),
      full2(Wb0T), full2(bb0), full2(Wb1T), full2(bb1), full2(Wb2T), full2(bb2),
      full2(Wt0T), full2(bt0), full2(Wt1T), full2(bt1), full2(Wt2T), full2(bt2),
      full2(Wt3T), full2(bt3), full2(Wt4T), full2(bt4),
  ]
  return pl.pallas_call(
      _tc_body,
      grid=(grid,),
      in_specs=in_specs,
      out_specs=pl.BlockSpec((1, block_b), lambda i: (0, i)),
      out_shape=jax.ShapeDtypeStruct((1, B), jnp.float32),
      scratch_shapes=[
          pltpu.VMEM((Z_PAD, block_b), jnp.float32),
      ],
  )(num_T, pairs, Wb0T, bb0, Wb1T, bb1, Wb2T, bb2,
    Wt0T, bt0, Wt1T, bt1, Wt2T, bt2, Wt3T, bt3, Wt4T, bt4)


def kernel(numerical_features, categorical_features, embedding_tables,
           Wb0, bb0, Wb1, bb1, Wb2, bb2,
           Wt0, bt0, Wt1, bt1, Wt2, bt2, Wt3, bt3, Wt4, bt4):
  tab_T = embedding_tables.transpose(0, 2, 1)
  idx_T = categorical_features.T

  pairs = _sc_gather(tab_T, idx_T)

  Wt0p = jnp.concatenate([Wt0, jnp.zeros((1, Wt0.shape[1]), Wt0.dtype)], axis=0)
  col = lambda b: b[:, None]
  out_T = _tc_dense(
      numerical_features.T, pairs,
      Wb0.T, col(bb0), Wb1.T, col(bb1), Wb2.T, col(bb2),
      Wt0p.T, col(bt0), Wt1.T, col(bt1), Wt2.T, col(bt2),
      Wt3.T, col(bt3), Wt4.T, col(bt4),
      block_b=256)
  return out_T.T
